# initial kernel scaffold (unmeasured)
import jax
import jax.numpy as jnp
from jax import lax
from jax.experimental import pallas as pl
from jax.experimental.pallas import tpu as pltpu

N_DEV = 32


def kernel(x, w_mat, scale_x, scale_w):
    m, k = x.shape
    _, n = w_mat.shape
    ch = m // N_DEV

    def body(x_ref, w_ref, sx_ref, sw_ref, out_ref, comm_ref,
             rs_send_sem, rs_recv_sem, ag_send_sem, ag_recv_sem,
             rs_credit, ag_credit):
        me = lax.axis_index("i")
        left = (me - 1) % N_DEV
        right = (me + 1) % N_DEV

        out_ref[...] = jnp.dot(
            x_ref[...], w_ref[...], preferred_element_type=jnp.float32
        )

        barrier_sem = pltpu.get_barrier_semaphore()
        for nbr in (left, right):
            pl.semaphore_signal(
                barrier_sem, inc=1,
                device_id=(nbr,), device_id_type=pl.DeviceIdType.MESH,
            )
        pl.semaphore_wait(barrier_sem, 2)

        for s in range(N_DEV - 1):
            send_idx = (me - s) % N_DEV
            recv_idx = (me - s - 1) % N_DEV
            if s > 0:
                pl.semaphore_wait(rs_credit, 1)
            rdma = pltpu.make_async_remote_copy(
                src_ref=out_ref.at[pl.ds(send_idx * ch, ch), :],
                dst_ref=comm_ref,
                send_sem=rs_send_sem,
                recv_sem=rs_recv_sem,
                device_id=(right,),
                device_id_type=pl.DeviceIdType.MESH,
            )
            rdma.start()
            rdma.wait()
            out_ref[pl.ds(recv_idx * ch, ch), :] = (
                out_ref[pl.ds(recv_idx * ch, ch), :] + comm_ref[...]
            )
            if s < N_DEV - 2:
                pl.semaphore_signal(
                    rs_credit, inc=1,
                    device_id=(left,), device_id_type=pl.DeviceIdType.MESH,
                )

        own = (me + 1) % N_DEV
        scale = sx_ref[0] * sw_ref[0]
        out_ref[pl.ds(own * ch, ch), :] = jnp.maximum(
            out_ref[pl.ds(own * ch, ch), :] * scale, 0.0
        )

        for s in range(N_DEV - 1):
            c = (me + 1 - s) % N_DEV
            if s > 0:
                pl.semaphore_wait(ag_credit, 1)
            rdma = pltpu.make_async_remote_copy(
                src_ref=out_ref.at[pl.ds(c * ch, ch), :],
                dst_ref=out_ref.at[pl.ds(c * ch, ch), :],
                send_sem=ag_send_sem,
                recv_sem=ag_recv_sem,
                device_id=(right,),
                device_id_type=pl.DeviceIdType.MESH,
            )
            rdma.start()
            rdma.wait()
            if s < N_DEV - 2:
                pl.semaphore_signal(
                    ag_credit, inc=1,
                    device_id=(left,), device_id_type=pl.DeviceIdType.MESH,
                )

    return pl.pallas_call(
        body,
        out_shape=jax.ShapeDtypeStruct((m, n), jnp.float32),
        in_specs=[
            pl.BlockSpec(memory_space=pltpu.VMEM),
            pl.BlockSpec(memory_space=pltpu.VMEM),
            pl.BlockSpec(memory_space=pltpu.SMEM),
            pl.BlockSpec(memory_space=pltpu.SMEM),
        ],
        out_specs=pl.BlockSpec(memory_space=pltpu.VMEM),
        scratch_shapes=[
            pltpu.VMEM((ch, n), jnp.float32),
            pltpu.SemaphoreType.DMA,
            pltpu.SemaphoreType.DMA,
            pltpu.SemaphoreType.DMA,
            pltpu.SemaphoreType.DMA,
            pltpu.SemaphoreType.REGULAR,
            pltpu.SemaphoreType.REGULAR,
        ],
        compiler_params=pltpu.CompilerParams(collective_id=0),
    )(x, w_mat, scale_x, scale_w)


# baseline (device time: 1169425 ns/iter reference)
import jax
import jax.numpy as jnp
from jax import lax
from jax.experimental import pallas as pl
from jax.experimental.pallas import tpu as pltpu

N_DEV = 32


def kernel(x, w_mat, scale_x, scale_w):
    m, k = x.shape
    _, n = w_mat.shape
    ch = m // N_DEV

    def body(x_ref, w_ref, sx_ref, sw_ref, out_ref, comm_ref,
             rs_send_sem, rs_recv_sem, ag_send_sem, ag_recv_sem,
             rs_credit, ag_credit):
        me = lax.axis_index("i")
        left = (me - 1) % N_DEV
        right = (me + 1) % N_DEV

        out_ref[...] = jnp.dot(
            x_ref[...], w_ref[...], preferred_element_type=jnp.float32
        )

        barrier_sem = pltpu.get_barrier_semaphore()
        for nbr in (left, right):
            pl.semaphore_signal(
                barrier_sem, inc=1,
                device_id=(nbr,), device_id_type=pl.DeviceIdType.MESH,
            )
        pl.semaphore_wait(barrier_sem, 2)

        for s in range(N_DEV - 1):
            send_idx = (me - s) % N_DEV
            recv_idx = (me - s - 1) % N_DEV
            if s > 0:
                pl.semaphore_wait(rs_credit, 1)
            rdma = pltpu.make_async_remote_copy(
                src_ref=out_ref.at[pl.ds(send_idx * ch, ch), :],
                dst_ref=comm_ref,
                send_sem=rs_send_sem,
                recv_sem=rs_recv_sem,
                device_id=(right,),
                device_id_type=pl.DeviceIdType.MESH,
            )
            rdma.start()
            rdma.wait()
            out_ref[pl.ds(recv_idx * ch, ch), :] = (
                out_ref[pl.ds(recv_idx * ch, ch), :] + comm_ref[...]
            )
            if s < N_DEV - 2:
                pl.semaphore_signal(
                    rs_credit, inc=1,
                    device_id=(left,), device_id_type=pl.DeviceIdType.MESH,
                )

        own = (me + 1) % N_DEV
        scale = sx_ref[0] * sw_ref[0]
        out_ref[pl.ds(own * ch, ch), :] = jnp.maximum(
            out_ref[pl.ds(own * ch, ch), :] * scale, 0.0
        )

        for s in range(N_DEV - 1):
            c = (me + 1 - s) % N_DEV
            if s > 0:
                pl.semaphore_wait(ag_credit, 1)
            rdma = pltpu.make_async_remote_copy(
                src_ref=out_ref.at[pl.ds(c * ch, ch), :],
                dst_ref=out_ref.at[pl.ds(c * ch, ch), :],
                send_sem=ag_send_sem,
                recv_sem=ag_recv_sem,
                device_id=(right,),
                device_id_type=pl.DeviceIdType.MESH,
            )
            rdma.start()
            rdma.wait()
            if s < N_DEV - 2:
                pl.semaphore_signal(
                    ag_credit, inc=1,
                    device_id=(left,), device_id_type=pl.DeviceIdType.MESH,
                )

    return pl.pallas_call(
        body,
        out_shape=jax.ShapeDtypeStruct((m, n), jnp.float32),
        in_specs=[
            pl.BlockSpec(memory_space=pltpu.VMEM),
            pl.BlockSpec(memory_space=pltpu.VMEM),
            pl.BlockSpec(memory_space=pltpu.SMEM),
            pl.BlockSpec(memory_space=pltpu.SMEM),
        ],
        out_specs=pl.BlockSpec(memory_space=pltpu.VMEM),
        scratch_shapes=[
            pltpu.VMEM((ch, n), jnp.float32),
            pltpu.SemaphoreType.DMA,
            pltpu.SemaphoreType.DMA,
            pltpu.SemaphoreType.DMA,
            pltpu.SemaphoreType.DMA,
            pltpu.SemaphoreType.REGULAR,
            pltpu.SemaphoreType.REGULAR,
        ],
        compiler_params=pltpu.CompilerParams(
            collective_id=0, vmem_limit_bytes=100 * 1024 * 1024
        ),
    )(x, w_mat, scale_x, scale_w)


# device time: 872523 ns/iter; 1.3403x vs baseline; 1.3403x over previous
import jax
import jax.numpy as jnp
from jax import lax
from jax.experimental import pallas as pl
from jax.experimental.pallas import tpu as pltpu

N_DEV = 32


def kernel(x, w_mat, scale_x, scale_w):
    m, k = x.shape
    _, n = w_mat.shape
    ch = m // N_DEV
    ha = n // 2

    def body(x_ref, w_ref, sx_ref, sw_ref, out_ref, comm_a, comm_b,
             sa_send, sa_recv, sb_send, sb_recv,
             ga_send, ga_recv, gb_send, gb_recv,
             cr_a, cr_b, cg_a, cg_b):
        me = lax.axis_index("i")
        left = (me - 1) % N_DEV
        right = (me + 1) % N_DEV

        out_ref[...] = jnp.dot(
            x_ref[...], w_ref[...], preferred_element_type=jnp.float32
        )

        barrier_sem = pltpu.get_barrier_semaphore()
        for nbr in (left, right):
            pl.semaphore_signal(
                barrier_sem, inc=1,
                device_id=(nbr,), device_id_type=pl.DeviceIdType.MESH,
            )
        pl.semaphore_wait(barrier_sem, 2)

        for s in range(N_DEV - 1):
            ia_s = (me - s) % N_DEV
            ia_r = (me - s - 1) % N_DEV
            ib_s = (me + s) % N_DEV
            ib_r = (me + s + 1) % N_DEV
            if s > 0:
                pl.semaphore_wait(cr_a, 1)
                pl.semaphore_wait(cr_b, 1)
            rdma_a = pltpu.make_async_remote_copy(
                src_ref=out_ref.at[pl.ds(ia_s * ch, ch), 0:ha],
                dst_ref=comm_a,
                send_sem=sa_send, recv_sem=sa_recv,
                device_id=(right,), device_id_type=pl.DeviceIdType.MESH,
            )
            rdma_b = pltpu.make_async_remote_copy(
                src_ref=out_ref.at[pl.ds(ib_s * ch, ch), ha:n],
                dst_ref=comm_b,
                send_sem=sb_send, recv_sem=sb_recv,
                device_id=(left,), device_id_type=pl.DeviceIdType.MESH,
            )
            rdma_a.start()
            rdma_b.start()
            rdma_a.wait()
            out_ref[pl.ds(ia_r * ch, ch), 0:ha] = (
                out_ref[pl.ds(ia_r * ch, ch), 0:ha] + comm_a[...]
            )
            rdma_b.wait()
            out_ref[pl.ds(ib_r * ch, ch), ha:n] = (
                out_ref[pl.ds(ib_r * ch, ch), ha:n] + comm_b[...]
            )
            if s < N_DEV - 2:
                pl.semaphore_signal(
                    cr_a, inc=1,
                    device_id=(left,), device_id_type=pl.DeviceIdType.MESH,
                )
                pl.semaphore_signal(
                    cr_b, inc=1,
                    device_id=(right,), device_id_type=pl.DeviceIdType.MESH,
                )

        own_a = (me + 1) % N_DEV
        own_b = (me - 1) % N_DEV
        scale = sx_ref[0] * sw_ref[0]
        out_ref[pl.ds(own_a * ch, ch), 0:ha] = jnp.maximum(
            out_ref[pl.ds(own_a * ch, ch), 0:ha] * scale, 0.0
        )
        out_ref[pl.ds(own_b * ch, ch), ha:n] = jnp.maximum(
            out_ref[pl.ds(own_b * ch, ch), ha:n] * scale, 0.0
        )

        for s in range(N_DEV - 1):
            ca = (me + 1 - s) % N_DEV
            cb = (me - 1 + s) % N_DEV
            if s > 0:
                pl.semaphore_wait(cg_a, 1)
                pl.semaphore_wait(cg_b, 1)
            rdma_a = pltpu.make_async_remote_copy(
                src_ref=out_ref.at[pl.ds(ca * ch, ch), 0:ha],
                dst_ref=out_ref.at[pl.ds(ca * ch, ch), 0:ha],
                send_sem=ga_send, recv_sem=ga_recv,
                device_id=(right,), device_id_type=pl.DeviceIdType.MESH,
            )
            rdma_b = pltpu.make_async_remote_copy(
                src_ref=out_ref.at[pl.ds(cb * ch, ch), ha:n],
                dst_ref=out_ref.at[pl.ds(cb * ch, ch), ha:n],
                send_sem=gb_send, recv_sem=gb_recv,
                device_id=(left,), device_id_type=pl.DeviceIdType.MESH,
            )
            rdma_a.start()
            rdma_b.start()
            rdma_a.wait()
            rdma_b.wait()
            if s < N_DEV - 2:
                pl.semaphore_signal(
                    cg_a, inc=1,
                    device_id=(left,), device_id_type=pl.DeviceIdType.MESH,
                )
                pl.semaphore_signal(
                    cg_b, inc=1,
                    device_id=(right,), device_id_type=pl.DeviceIdType.MESH,
                )

    return pl.pallas_call(
        body,
        out_shape=jax.ShapeDtypeStruct((m, n), jnp.float32),
        in_specs=[
            pl.BlockSpec(memory_space=pltpu.VMEM),
            pl.BlockSpec(memory_space=pltpu.VMEM),
            pl.BlockSpec(memory_space=pltpu.SMEM),
            pl.BlockSpec(memory_space=pltpu.SMEM),
        ],
        out_specs=pl.BlockSpec(memory_space=pltpu.VMEM),
        scratch_shapes=[
            pltpu.VMEM((ch, ha), jnp.float32),
            pltpu.VMEM((ch, ha), jnp.float32),
            pltpu.SemaphoreType.DMA,
            pltpu.SemaphoreType.DMA,
            pltpu.SemaphoreType.DMA,
            pltpu.SemaphoreType.DMA,
            pltpu.SemaphoreType.DMA,
            pltpu.SemaphoreType.DMA,
            pltpu.SemaphoreType.DMA,
            pltpu.SemaphoreType.DMA,
            pltpu.SemaphoreType.REGULAR,
            pltpu.SemaphoreType.REGULAR,
            pltpu.SemaphoreType.REGULAR,
            pltpu.SemaphoreType.REGULAR,
        ],
        compiler_params=pltpu.CompilerParams(
            collective_id=0, vmem_limit_bytes=100 * 1024 * 1024
        ),
    )(x, w_mat, scale_x, scale_w)
